# baseline (speedup 1.0000x reference)
import functools

import jax
import jax.numpy as jnp
from jax import lax
from jax.experimental import pallas as pl
from jax.experimental.pallas import tpu as pltpu
from jax.experimental.pallas import tpu_sc as plsc


def _mm3_body(a, b, c, wa, wb, wc, bias, o):
    acc = jnp.dot(a[...], wa[...], preferred_element_type=jnp.float32)
    acc += jnp.dot(b[...], wb[...], preferred_element_type=jnp.float32)
    acc += jnp.dot(c[...], wc[...], preferred_element_type=jnp.float32)
    o[...] = acc + bias[...]


def _mm3(a, b, c, wa, wb, wc, bias, block_rows):
    n, d = a.shape
    do = wa.shape[1]
    row_spec = pl.BlockSpec((block_rows, d), lambda i: (i, 0))
    w_spec = pl.BlockSpec((d, do), lambda i: (0, 0))
    return pl.pallas_call(
        _mm3_body,
        grid=(n // block_rows,),
        in_specs=[row_spec, row_spec, row_spec, w_spec, w_spec, w_spec,
                  pl.BlockSpec((1, do), lambda i: (0, 0))],
        out_specs=pl.BlockSpec((block_rows, do), lambda i: (i, 0)),
        out_shape=jax.ShapeDtypeStruct((n, do), jnp.float32),
    )(a, b, c, wa, wb, wc, bias.reshape(1, do))


def _relu_mm_body(x, w, bias, o):
    o[...] = jax.nn.relu(
        jnp.dot(x[...], w[...], preferred_element_type=jnp.float32) + bias[...])


def _relu_mm(x, w, bias, block_rows):
    n, d = x.shape
    do = w.shape[1]
    return pl.pallas_call(
        _relu_mm_body,
        grid=(n // block_rows,),
        in_specs=[pl.BlockSpec((block_rows, d), lambda i: (i, 0)),
                  pl.BlockSpec((d, do), lambda i: (0, 0)),
                  pl.BlockSpec((1, do), lambda i: (0, 0))],
        out_specs=pl.BlockSpec((block_rows, do), lambda i: (i, 0)),
        out_shape=jax.ShapeDtypeStruct((n, do), jnp.float32),
    )(x, w, bias.reshape(1, do))


_NW = 32
_DAUG = 80


def _make_segsum(n_rows_padded, k_chunks, n_table):
    mesh = plsc.VectorSubcoreMesh(core_axis_name="c", subcore_axis_name="s")
    rows_per_sub = n_rows_padded // 16

    @functools.partial(
        pl.kernel, mesh=mesh,
        compiler_params=pltpu.CompilerParams(use_tc_tiling_on_sc=False),
        out_type=jax.ShapeDtypeStruct((2, n_rows_padded, _DAUG), jnp.float32),
        scratch_types=[
            pltpu.VMEM((k_chunks, 128), jnp.int32),
            pltpu.VMEM((k_chunks, 128), jnp.int32),
            pltpu.VMEM((2, 128, _DAUG), jnp.float32),
            pltpu.VMEM_SHARED((n_rows_padded, _DAUG), jnp.float32),
            pltpu.SemaphoreType.DMA,
            pltpu.SemaphoreType.DMA,
        ],
    )
    def seg_sum(table_hbm, src_hbm, dst_hbm, out_hbm,
                src_v, dst_v, rows_v, acc_sh, gsem, ssem):
        cid = lax.axis_index("c")
        sid = lax.axis_index("s")

        zrow = jnp.zeros((16,), jnp.float32)

        def zero_body(i, _):
            for q in range(_DAUG // 16):
                rows_v[0, i, pl.ds(q * 16, 16)] = zrow
            return 0

        lax.fori_loop(0, 128, zero_body, 0)
        base = sid * rows_per_sub
        nfull = rows_per_sub // 128
        for z in range(nfull):
            pltpu.sync_copy(rows_v.at[0],
                            acc_sh.at[pl.ds(base + z * 128, 128)])
        rem = rows_per_sub - nfull * 128
        if rem:
            pltpu.sync_copy(rows_v.at[0, pl.ds(0, rem)],
                            acc_sh.at[pl.ds(base + nfull * 128, rem)])
        plsc.subcore_barrier()

        pltpu.sync_copy(src_hbm.at[sid], src_v)
        pltpu.sync_copy(dst_hbm.at[sid], dst_v)
        off = (cid * n_table).astype(jnp.int32)

        def bump_body(i, _):
            r = lax.div(i, jnp.int32(8))
            q = lax.rem(i, jnp.int32(8))
            src_v[r, pl.ds(q * 16, 16)] = src_v[r, pl.ds(q * 16, 16)] + off
            return 0

        lax.fori_loop(0, k_chunks * 8, bump_body, 0)

        def chunk_body(j, _):
            slot = lax.rem(j, 2)
            pltpu.async_copy(table_hbm.at[src_v.at[j]], rows_v.at[slot],
                             gsem).wait()
            pltpu.sync_copy(rows_v.at[slot], acc_sh.at[dst_v.at[j]],
                            add=True)
            return 0

        lax.fori_loop(0, k_chunks, chunk_body, 0)
        plsc.subcore_barrier()

        for z in range(nfull):
            pltpu.sync_copy(acc_sh.at[pl.ds(base + z * 128, 128)],
                            rows_v.at[1])
            pltpu.sync_copy(rows_v.at[1],
                            out_hbm.at[cid, pl.ds(base + z * 128, 128)])
        if rem:
            pltpu.sync_copy(acc_sh.at[pl.ds(base + nfull * 128, rem)],
                            rows_v.at[1, pl.ds(0, rem)])
            pltpu.sync_copy(rows_v.at[1, pl.ds(0, rem)],
                            out_hbm.at[cid, pl.ds(base + nfull * 128, rem)])

    return seg_sum


_R = 1568
_CS = 2048
_CAP = 3584
_GB = 64


def _make_segmax(n_table, n_chunks):
    mesh = plsc.VectorSubcoreMesh(core_axis_name="c", subcore_axis_name="s")

    gdn = lax.GatherDimensionNumbers(
        offset_dims=(), collapsed_slice_dims=(0,), start_index_map=(0,))

    def _dg(v, idx):
        return lax.gather(v, idx[:, None], gdn, (1,),
                          mode=lax.GatherScatterMode.PROMISE_IN_BOUNDS)

    @functools.partial(
        pl.kernel, mesh=mesh,
        compiler_params=pltpu.CompilerParams(use_tc_tiling_on_sc=False,
                                             needs_layout_passes=False),
        out_type=jax.ShapeDtypeStruct((32 * _R, 64), jnp.int32),
        scratch_types=[
            pltpu.VMEM((_R, 64), jnp.int32),
            pltpu.VMEM((_CS,), jnp.int32),
            pltpu.VMEM((_CS,), jnp.int32),
            pltpu.VMEM((_CS,), jnp.float32),
            pltpu.VMEM((_CAP + 16,), jnp.int32),
            pltpu.VMEM((_CAP + 16,), jnp.int32),
            pltpu.VMEM((_CAP + 16,), jnp.float32),
            pltpu.VMEM((3, 16), jnp.int32),
            pltpu.VMEM((16,), jnp.int32),
            pltpu.VMEM((2, _GB, 64), jnp.int32),
            pltpu.SMEM((2,), jnp.int32),
            pltpu.SemaphoreType.DMA,
        ],
    )
    def seg_max(table_hbm, key_hbm, gidx_hbm, ew_hbm, acc_out,
                acc_v, keyb, gixb, ewb, glist, dlist, ewlist,
                pbuf, mbuf, rows_v, cnts, gsem):
        cid = lax.axis_index("c")
        sid = lax.axis_index("s")
        wid = cid * 16 + sid
        base = wid * _R

        def init_body(i, _):
            neg1 = jnp.full((16,), -1082081408, jnp.int32)
            for q in range(4):
                acc_v[i, pl.ds(q * 16, 16)] = neg1
            return 0

        lax.fori_loop(0, _R, init_body, 0)

        def zlist_body(i, _):
            glist[pl.ds(i * 16, 16)] = jnp.zeros((16,), jnp.int32)
            return 0

        lax.fori_loop(0, (_CAP + 16) // 16, zlist_body, 0)
        pbuf[0, :] = jnp.zeros((16,), jnp.int32)
        pbuf[1, :] = jnp.zeros((16,), jnp.int32)
        pbuf[2, :] = jnp.zeros((16,), jnp.int32)
        cnts[0] = 0
        cnts[1] = 0

        def drain(n_matched):
            nb = lax.div(n_matched + (_GB - 1), jnp.int32(_GB))

            def batch_body(b, _):
                slot = lax.rem(b, 2)
                pltpu.async_copy(
                    table_hbm.at[glist.at[pl.ds(b * _GB, _GB)]],
                    rows_v.at[slot], gsem).wait()
                for g in range(_GB // 16):
                    j0 = b * _GB + g * 16

                    @pl.when(j0 < n_matched)
                    def _(j0=j0, g=g, slot=slot):
                        dvec = dlist[pl.ds(j0, 16)]
                        evec = ewlist[pl.ds(j0, 16)]
                        for l in range(16):

                            @pl.when(j0 + l < n_matched)
                            def _(l=l, dvec=dvec, evec=evec, g=g, slot=slot):
                                dloc = dvec[l]
                                s32 = jnp.broadcast_to(evec[l], (16,))
                                scale = plsc.pack(
                                    s32, s32,
                                    format=plsc.PackFormat.INTERLEAVED)
                                jl = g * 16 + l
                                for q in range(4):
                                    a = acc_v[dloc, pl.ds(q * 16, 16)]
                                    mi = rows_v[slot, jl, pl.ds(q * 16, 16)]
                                    ab = plsc.bitcast(a, jnp.bfloat16)
                                    mb = plsc.bitcast(mi, jnp.bfloat16)
                                    nv = jnp.maximum(ab, mb * scale)
                                    acc_v[dloc, pl.ds(q * 16, 16)] = (
                                        plsc.bitcast(nv, jnp.int32))
                return 0

            lax.fori_loop(0, nb, batch_body, 0)

        def flush_pending():
            ns = cnts[0]
            glist[pl.ds(ns * 16, 16)] = pbuf[0, :]
            dlist[pl.ds(ns * 16, 16)] = pbuf[1, :]
            ewlist[pl.ds(ns * 16, 16)] = plsc.bitcast(pbuf[2, :], jnp.float32)

        def chunk_body(c, _):
            pltpu.sync_copy(key_hbm.at[pl.ds(c * _CS, _CS)], keyb)
            pltpu.sync_copy(gidx_hbm.at[pl.ds(c * _CS, _CS)], gixb)
            pltpu.sync_copy(ew_hbm.at[pl.ds(c * _CS, _CS)], ewb)

            def vreg_body(i, _):
                k = keyb[pl.ds(i * 16, 16)]
                local = k - base
                inb = (local >= 0) & (local < _R)
                m = plsc.all_reduce_population_count(inb)[0]

                @pl.when(m > 0)
                def _():
                    lanes = lax.iota(jnp.int32, 16)
                    gv = gixb[pl.ds(i * 16, 16)]
                    ev = plsc.bitcast(ewb[pl.ds(i * 16, 16)], jnp.int32)
                    mbuf[...] = jnp.where(inb, jnp.ones((16,), jnp.int32),
                                          jnp.zeros((16,), jnp.int32))

                    def edge_body(t, _):
                        p = cnts[1]

                        @pl.when(p >= 16)
                        def _():
                            ns = cnts[0]
                            glist[pl.ds(ns * 16, 16)] = pbuf[0, :]
                            dlist[pl.ds(ns * 16, 16)] = pbuf[1, :]
                            ewlist[pl.ds(ns * 16, 16)] = plsc.bitcast(
                                pbuf[2, :], jnp.float32)
                            cnts[0] = ns + 1
                            cnts[1] = 0

                        p2 = cnts[1]
                        mk = mbuf[...]
                        ls = plsc.all_reduce_ffs(mk > 0)
                        mbuf[...] = jnp.where(lanes == ls,
                                              jnp.zeros((16,), jnp.int32), mk)
                        wm = lanes == jnp.broadcast_to(p2, (16,))
                        pbuf[0, :] = jnp.where(wm, _dg(gv, ls), pbuf[0, :])
                        pbuf[1, :] = jnp.where(wm, _dg(local, ls), pbuf[1, :])
                        pbuf[2, :] = jnp.where(wm, _dg(ev, ls), pbuf[2, :])
                        cnts[1] = p2 + 1
                        return 0

                    lax.fori_loop(0, m, edge_body, 0)

                return 0

            lax.fori_loop(0, _CS // 16, vreg_body, 0)
            n_matched = cnts[0] * 16 + cnts[1]
            do_drain = n_matched >= (_CAP - _CS)

            @pl.when(do_drain)
            def _():
                flush_pending()
                drain(n_matched)
                cnts[0] = 0
                cnts[1] = 0

            return 0

        lax.fori_loop(0, n_chunks, chunk_body, 0)
        flush_pending()
        drain(cnts[0] * 16 + cnts[1])

        pltpu.sync_copy(acc_v, acc_out.at[pl.ds(base, _R)])

    return seg_max


def _pack_bf16(x):
    b = x.astype(jnp.bfloat16)
    return jax.lax.bitcast_convert_type(
        b.reshape(x.shape[0], x.shape[1] // 2, 2), jnp.int32)


def _unpack_bf16(p):
    b = jax.lax.bitcast_convert_type(p, jnp.bfloat16)
    return b.reshape(p.shape[0], p.shape[1] * 2)


def _segmax(table_f32, key, gidx, ew, n_out):
    e = key.shape[0]
    n_chunks = -(-e // _CS)
    e_pad = n_chunks * _CS
    pad = e_pad - e
    key_p = jnp.concatenate([key, jnp.full((pad,), 2**30, jnp.int32)])
    gidx_p = jnp.concatenate([gidx, jnp.zeros((pad,), jnp.int32)])
    ew_p = jnp.concatenate([ew, jnp.zeros((pad,), jnp.float32)])
    table = _pack_bf16(table_f32)
    acc = _make_segmax(table.shape[0], n_chunks)(table, key_p, gidx_p, ew_p)
    return jax.nn.relu(_unpack_bf16(acc)[:n_out].astype(jnp.float32))


def kernel(node_feat, net_feat, pin_feat, edge_feat, gc_W, gc_b, t_pool_W, t_pool_b,
           t_neigh_W, t_self_W, t_self_b, g_pool_W, g_pool_b, g_neigh_W, g_self_W,
           g_self_b, topo_w_W, topo_w_b, geom_w_W, geom_w_b, net_lin_W, net_lin_b,
           pin_src, pin_dst, near_src, near_dst):
    N_CELL, _ = node_feat.shape
    N_NET, _ = net_feat.shape
    E_PIN = pin_src.shape[0]
    E_NEAR = near_src.shape[0]

    ew_pin = jax.nn.sigmoid(pin_feat @ topo_w_W + topo_w_b)
    ew_near = jax.nn.sigmoid(edge_feat @ geom_w_W + geom_w_b)

    h_pool = _relu_mm(net_feat, t_pool_W, t_pool_b, 1000)
    neigh = _segmax(h_pool, pin_src, pin_dst, ew_pin[:, 0], N_CELL)

    h_pool2 = _relu_mm(node_feat, g_pool_W, g_pool_b, 1000)
    neigh2 = _segmax(h_pool2, near_dst, near_src, ew_near[:, 0], N_CELL)

    deg_src = jax.ops.segment_sum(jnp.ones((E_PIN,), jnp.float32),
                                  pin_src, N_CELL)
    norm_src = jnp.where(deg_src > 0, deg_src ** -0.5, 0.0)
    feat = node_feat * norm_src[:, None]

    n_net_pad = 10112
    k_chunks = -(-E_PIN // (16 * 128))
    e_pad = 16 * k_chunks * 128
    src_p = jnp.concatenate(
        [pin_src, jnp.zeros((e_pad - E_PIN,), jnp.int32)]).reshape(16, k_chunks, 128)
    dst_p = jnp.concatenate(
        [pin_dst, jnp.full((e_pad - E_PIN,), n_net_pad - 1, jnp.int32)]
    ).reshape(16, k_chunks, 128)
    zc = jnp.zeros((N_CELL, _DAUG - 65), jnp.float32)
    table = jnp.concatenate([
        jnp.concatenate([feat[:, :64], jnp.ones((N_CELL, 1), jnp.float32), zc], 1),
        jnp.concatenate([feat[:, 64:], jnp.zeros((N_CELL, 1), jnp.float32), zc], 1),
    ], axis=0)
    parts = _make_segsum(n_net_pad, k_chunks, N_CELL)(table, src_p, dst_p)
    agg = jnp.concatenate([parts[0, :N_NET, :64], parts[1, :N_NET, :64]], axis=1)
    deg_dst = parts[0, :N_NET, 64]
    norm_dst = jnp.where(deg_dst > 0, deg_dst ** -0.5, 0.0)

    cell_out = _mm3(node_feat, neigh, neigh2,
                    t_self_W + g_self_W, t_neigh_W, g_neigh_W,
                    t_self_b + g_self_b, 1000)

    aggn = agg * norm_dst[:, None]
    net_out = _mm3(aggn, net_feat, jnp.zeros_like(net_feat),
                   gc_W, net_lin_W, jnp.zeros_like(net_lin_W),
                   gc_b + net_lin_b, 1000)
    return (cell_out, net_out)


# speedup vs baseline: 1.1667x; 1.1667x over previous
import functools

import jax
import jax.numpy as jnp
from jax import lax
from jax.experimental import pallas as pl
from jax.experimental.pallas import tpu as pltpu
from jax.experimental.pallas import tpu_sc as plsc


def _mm3_body(a, b, c, wa, wb, wc, bias, o):
    acc = jnp.dot(a[...], wa[...], preferred_element_type=jnp.float32)
    acc += jnp.dot(b[...], wb[...], preferred_element_type=jnp.float32)
    acc += jnp.dot(c[...], wc[...], preferred_element_type=jnp.float32)
    o[...] = acc + bias[...]


def _mm3(a, b, c, wa, wb, wc, bias, block_rows):
    n, d = a.shape
    do = wa.shape[1]
    row_spec = pl.BlockSpec((block_rows, d), lambda i: (i, 0))
    w_spec = pl.BlockSpec((d, do), lambda i: (0, 0))
    return pl.pallas_call(
        _mm3_body,
        grid=(n // block_rows,),
        in_specs=[row_spec, row_spec, row_spec, w_spec, w_spec, w_spec,
                  pl.BlockSpec((1, do), lambda i: (0, 0))],
        out_specs=pl.BlockSpec((block_rows, do), lambda i: (i, 0)),
        out_shape=jax.ShapeDtypeStruct((n, do), jnp.float32),
    )(a, b, c, wa, wb, wc, bias.reshape(1, do))


def _relu_mm_body(x, w, bias, o):
    o[...] = jax.nn.relu(
        jnp.dot(x[...], w[...], preferred_element_type=jnp.float32) + bias[...])


def _relu_mm(x, w, bias, block_rows):
    n, d = x.shape
    do = w.shape[1]
    return pl.pallas_call(
        _relu_mm_body,
        grid=(n // block_rows,),
        in_specs=[pl.BlockSpec((block_rows, d), lambda i: (i, 0)),
                  pl.BlockSpec((d, do), lambda i: (0, 0)),
                  pl.BlockSpec((1, do), lambda i: (0, 0))],
        out_specs=pl.BlockSpec((block_rows, do), lambda i: (i, 0)),
        out_shape=jax.ShapeDtypeStruct((n, do), jnp.float32),
    )(x, w, bias.reshape(1, do))


_NW = 32
_DAUG = 80


def _make_segsum(n_rows_padded, k_chunks, n_table):
    mesh = plsc.VectorSubcoreMesh(core_axis_name="c", subcore_axis_name="s")
    rows_per_sub = n_rows_padded // 16

    @functools.partial(
        pl.kernel, mesh=mesh,
        compiler_params=pltpu.CompilerParams(use_tc_tiling_on_sc=False),
        out_type=jax.ShapeDtypeStruct((2, n_rows_padded, _DAUG), jnp.float32),
        scratch_types=[
            pltpu.VMEM((k_chunks, 128), jnp.int32),
            pltpu.VMEM((k_chunks, 128), jnp.int32),
            pltpu.VMEM((2, 128, _DAUG), jnp.float32),
            pltpu.VMEM_SHARED((n_rows_padded, _DAUG), jnp.float32),
            pltpu.SemaphoreType.DMA,
            pltpu.SemaphoreType.DMA,
        ],
    )
    def seg_sum(table_hbm, src_hbm, dst_hbm, out_hbm,
                src_v, dst_v, rows_v, acc_sh, gsem, ssem):
        cid = lax.axis_index("c")
        sid = lax.axis_index("s")

        zrow = jnp.zeros((16,), jnp.float32)

        def zero_body(i, _):
            for q in range(_DAUG // 16):
                rows_v[0, i, pl.ds(q * 16, 16)] = zrow
            return 0

        lax.fori_loop(0, 128, zero_body, 0)
        base = sid * rows_per_sub
        nfull = rows_per_sub // 128
        for z in range(nfull):
            pltpu.sync_copy(rows_v.at[0],
                            acc_sh.at[pl.ds(base + z * 128, 128)])
        rem = rows_per_sub - nfull * 128
        if rem:
            pltpu.sync_copy(rows_v.at[0, pl.ds(0, rem)],
                            acc_sh.at[pl.ds(base + nfull * 128, rem)])
        plsc.subcore_barrier()

        pltpu.sync_copy(src_hbm.at[sid], src_v)
        pltpu.sync_copy(dst_hbm.at[sid], dst_v)
        off = (cid * n_table).astype(jnp.int32)

        def bump_body(i, _):
            r = lax.div(i, jnp.int32(8))
            q = lax.rem(i, jnp.int32(8))
            src_v[r, pl.ds(q * 16, 16)] = src_v[r, pl.ds(q * 16, 16)] + off
            return 0

        lax.fori_loop(0, k_chunks * 8, bump_body, 0)

        def chunk_body(j, _):
            slot = lax.rem(j, 2)
            pltpu.async_copy(table_hbm.at[src_v.at[j]], rows_v.at[slot],
                             gsem).wait()
            pltpu.sync_copy(rows_v.at[slot], acc_sh.at[dst_v.at[j]],
                            add=True)
            return 0

        lax.fori_loop(0, k_chunks, chunk_body, 0)
        plsc.subcore_barrier()

        for z in range(nfull):
            pltpu.sync_copy(acc_sh.at[pl.ds(base + z * 128, 128)],
                            rows_v.at[1])
            pltpu.sync_copy(rows_v.at[1],
                            out_hbm.at[cid, pl.ds(base + z * 128, 128)])
        if rem:
            pltpu.sync_copy(acc_sh.at[pl.ds(base + nfull * 128, rem)],
                            rows_v.at[1, pl.ds(0, rem)])
            pltpu.sync_copy(rows_v.at[1, pl.ds(0, rem)],
                            out_hbm.at[cid, pl.ds(base + nfull * 128, rem)])

    return seg_sum


_R = 1568
_CS = 2048
_CAP = 3072
_GB = 64


def _make_segmax(n_table, n_chunks):
    mesh = plsc.VectorSubcoreMesh(core_axis_name="c", subcore_axis_name="s")

    gdn = lax.GatherDimensionNumbers(
        offset_dims=(), collapsed_slice_dims=(0,), start_index_map=(0,))

    def _dg(v, idx):
        return lax.gather(v, idx[:, None], gdn, (1,),
                          mode=lax.GatherScatterMode.PROMISE_IN_BOUNDS)

    @functools.partial(
        pl.kernel, mesh=mesh,
        compiler_params=pltpu.CompilerParams(use_tc_tiling_on_sc=False,
                                             needs_layout_passes=False),
        out_type=jax.ShapeDtypeStruct((32 * _R, 64), jnp.int32),
        scratch_types=[
            pltpu.VMEM((_R, 64), jnp.int32),
            pltpu.VMEM((2, _CS), jnp.int32),
            pltpu.VMEM((2, _CS), jnp.int32),
            pltpu.VMEM((2, _CS), jnp.float32),
            pltpu.VMEM((_CAP + 16,), jnp.int32),
            pltpu.VMEM((_CAP + 16,), jnp.int32),
            pltpu.VMEM((_CAP + 16,), jnp.float32),
            pltpu.VMEM((3, 16), jnp.int32),
            pltpu.VMEM((16,), jnp.int32),
            pltpu.VMEM((2, _GB, 64), jnp.int32),
            pltpu.SMEM((2,), jnp.int32),
            pltpu.SemaphoreType.DMA,
            pltpu.SemaphoreType.DMA,
            pltpu.SemaphoreType.DMA,
            pltpu.SemaphoreType.DMA,
        ],
    )
    def seg_max(table_hbm, key_hbm, gidx_hbm, ew_hbm, acc_out,
                acc_v, keyb, gixb, ewb, glist, dlist, ewlist,
                pbuf, mbuf, rows_v, cnts, csem0, csem1, gsem0, gsem1):
        cid = lax.axis_index("c")
        sid = lax.axis_index("s")
        wid = cid * 16 + sid
        base = wid * _R

        def init_body(i, _):
            neg1 = jnp.full((16,), -1082081408, jnp.int32)
            for q in range(4):
                acc_v[i, pl.ds(q * 16, 16)] = neg1
            return 0

        lax.fori_loop(0, _R, init_body, 0)

        def zlist_body(i, _):
            glist[pl.ds(i * 16, 16)] = jnp.zeros((16,), jnp.int32)
            return 0

        lax.fori_loop(0, (_CAP + 16) // 16, zlist_body, 0)
        pbuf[0, :] = jnp.zeros((16,), jnp.int32)
        pbuf[1, :] = jnp.zeros((16,), jnp.int32)
        pbuf[2, :] = jnp.zeros((16,), jnp.int32)
        cnts[0] = 0
        cnts[1] = 0

        def drain(n_matched):
            nb = lax.div(n_matched + (_GB - 1), jnp.int32(_GB))

            def fire_gather(b, slot):
                sem = [gsem0, gsem1][slot]
                pltpu.async_copy(
                    table_hbm.at[glist.at[pl.ds(b * _GB, _GB)]],
                    rows_v.at[slot], sem)

            @pl.when(nb > 0)
            def _():
                fire_gather(0, 0)

            def batch_body(b, _):
                slot = lax.rem(b, 2)

                for sl in range(2):
                    @pl.when((b + 1 < nb) & (slot == sl))
                    def _(sl=sl):
                        fire_gather(b + 1, 1 - sl)

                for sl in range(2):
                    @pl.when(slot == sl)
                    def _(sl=sl):
                        pltpu.make_async_copy(
                            table_hbm.at[glist.at[pl.ds(b * _GB, _GB)]],
                            rows_v.at[sl], [gsem0, gsem1][sl]).wait()
                for g in range(_GB // 16):
                    j0 = b * _GB + g * 16

                    @pl.when(j0 < n_matched)
                    def _(j0=j0, g=g, slot=slot):
                        dvec = dlist[pl.ds(j0, 16)]
                        evec = ewlist[pl.ds(j0, 16)]
                        for l in range(16):

                            @pl.when(j0 + l < n_matched)
                            def _(l=l, dvec=dvec, evec=evec, g=g, slot=slot):
                                dloc = dvec[l]
                                s32 = jnp.broadcast_to(evec[l], (16,))
                                scale = plsc.pack(
                                    s32, s32,
                                    format=plsc.PackFormat.INTERLEAVED)
                                jl = g * 16 + l
                                for q in range(4):
                                    a = acc_v[dloc, pl.ds(q * 16, 16)]
                                    mi = rows_v[slot, jl, pl.ds(q * 16, 16)]
                                    ab = plsc.bitcast(a, jnp.bfloat16)
                                    mb = plsc.bitcast(mi, jnp.bfloat16)
                                    nv = jnp.maximum(ab, mb * scale)
                                    acc_v[dloc, pl.ds(q * 16, 16)] = (
                                        plsc.bitcast(nv, jnp.int32))
                return 0

            lax.fori_loop(0, nb, batch_body, 0)

        def flush_pending():
            ns = cnts[0]
            glist[pl.ds(ns * 16, 16)] = pbuf[0, :]
            dlist[pl.ds(ns * 16, 16)] = pbuf[1, :]
            ewlist[pl.ds(ns * 16, 16)] = plsc.bitcast(pbuf[2, :], jnp.float32)

        def fire_chunk(c, slot):
            sem = [csem0, csem1][slot]
            pltpu.async_copy(key_hbm.at[pl.ds(c * _CS, _CS)],
                             keyb.at[slot], sem)
            pltpu.async_copy(gidx_hbm.at[pl.ds(c * _CS, _CS)],
                             gixb.at[slot], sem)
            pltpu.async_copy(ew_hbm.at[pl.ds(c * _CS, _CS)],
                             ewb.at[slot], sem)

        fire_chunk(0, 0)

        def chunk_body(c, _):
            cslot = lax.rem(c, 2)

            for sl in range(2):
                @pl.when((c + 1 < n_chunks) & (cslot == sl))
                def _(sl=sl):
                    fire_chunk(c + 1, 1 - sl)

            for sl in range(2):
                @pl.when(cslot == sl)
                def _(sl=sl):
                    sem = [csem0, csem1][sl]
                    pltpu.make_async_copy(
                        key_hbm.at[pl.ds(c * _CS, _CS)], keyb.at[sl],
                        sem).wait()
                    pltpu.make_async_copy(
                        gidx_hbm.at[pl.ds(c * _CS, _CS)], gixb.at[sl],
                        sem).wait()
                    pltpu.make_async_copy(
                        ew_hbm.at[pl.ds(c * _CS, _CS)], ewb.at[sl],
                        sem).wait()

            def vreg_body(i, _):
                k = keyb[cslot, pl.ds(i * 16, 16)]
                local = k - base
                inb = (local >= 0) & (local < _R)
                m = plsc.all_reduce_population_count(inb)[0]

                @pl.when(m > 0)
                def _():
                    lanes = lax.iota(jnp.int32, 16)
                    gv = gixb[cslot, pl.ds(i * 16, 16)]
                    ev = plsc.bitcast(ewb[cslot, pl.ds(i * 16, 16)], jnp.int32)
                    mbuf[...] = jnp.where(inb, jnp.ones((16,), jnp.int32),
                                          jnp.zeros((16,), jnp.int32))

                    def edge_body(t, _):
                        p = cnts[1]

                        @pl.when(p >= 16)
                        def _():
                            ns = cnts[0]
                            glist[pl.ds(ns * 16, 16)] = pbuf[0, :]
                            dlist[pl.ds(ns * 16, 16)] = pbuf[1, :]
                            ewlist[pl.ds(ns * 16, 16)] = plsc.bitcast(
                                pbuf[2, :], jnp.float32)
                            cnts[0] = ns + 1
                            cnts[1] = 0

                        p2 = cnts[1]
                        mk = mbuf[...]
                        ls = plsc.all_reduce_ffs(mk > 0)
                        mbuf[...] = jnp.where(lanes == ls,
                                              jnp.zeros((16,), jnp.int32), mk)
                        wm = lanes == jnp.broadcast_to(p2, (16,))
                        pbuf[0, :] = jnp.where(wm, _dg(gv, ls), pbuf[0, :])
                        pbuf[1, :] = jnp.where(wm, _dg(local, ls), pbuf[1, :])
                        pbuf[2, :] = jnp.where(wm, _dg(ev, ls), pbuf[2, :])
                        cnts[1] = p2 + 1
                        return 0

                    lax.fori_loop(0, m, edge_body, 0)

                return 0

            lax.fori_loop(0, _CS // 16, vreg_body, 0)
            n_matched = cnts[0] * 16 + cnts[1]
            do_drain = n_matched >= (_CAP - _CS)

            @pl.when(do_drain)
            def _():
                flush_pending()
                drain(n_matched)
                cnts[0] = 0
                cnts[1] = 0

            return 0

        lax.fori_loop(0, n_chunks, chunk_body, 0)
        flush_pending()
        drain(cnts[0] * 16 + cnts[1])

        pltpu.sync_copy(acc_v, acc_out.at[pl.ds(base, _R)])

    return seg_max


def _pack_bf16(x):
    b = x.astype(jnp.bfloat16)
    return jax.lax.bitcast_convert_type(
        b.reshape(x.shape[0], x.shape[1] // 2, 2), jnp.int32)


def _unpack_bf16(p):
    b = jax.lax.bitcast_convert_type(p, jnp.bfloat16)
    return b.reshape(p.shape[0], p.shape[1] * 2)


def _segmax(table_f32, key, gidx, ew, n_out):
    e = key.shape[0]
    n_chunks = -(-e // _CS)
    e_pad = n_chunks * _CS
    pad = e_pad - e
    key_p = jnp.concatenate([key, jnp.full((pad,), 2**30, jnp.int32)])
    gidx_p = jnp.concatenate([gidx, jnp.zeros((pad,), jnp.int32)])
    ew_p = jnp.concatenate([ew, jnp.zeros((pad,), jnp.float32)])
    table = _pack_bf16(table_f32)
    acc = _make_segmax(table.shape[0], n_chunks)(table, key_p, gidx_p, ew_p)
    return jax.nn.relu(_unpack_bf16(acc)[:n_out].astype(jnp.float32))


def kernel(node_feat, net_feat, pin_feat, edge_feat, gc_W, gc_b, t_pool_W, t_pool_b,
           t_neigh_W, t_self_W, t_self_b, g_pool_W, g_pool_b, g_neigh_W, g_self_W,
           g_self_b, topo_w_W, topo_w_b, geom_w_W, geom_w_b, net_lin_W, net_lin_b,
           pin_src, pin_dst, near_src, near_dst):
    N_CELL, _ = node_feat.shape
    N_NET, _ = net_feat.shape
    E_PIN = pin_src.shape[0]
    E_NEAR = near_src.shape[0]

    ew_pin = jax.nn.sigmoid(pin_feat @ topo_w_W + topo_w_b)
    ew_near = jax.nn.sigmoid(edge_feat @ geom_w_W + geom_w_b)

    h_pool = _relu_mm(net_feat, t_pool_W, t_pool_b, 1000)
    neigh = _segmax(h_pool, pin_src, pin_dst, ew_pin[:, 0], N_CELL)

    h_pool2 = _relu_mm(node_feat, g_pool_W, g_pool_b, 1000)
    neigh2 = _segmax(h_pool2, near_dst, near_src, ew_near[:, 0], N_CELL)

    deg_src = jax.ops.segment_sum(jnp.ones((E_PIN,), jnp.float32),
                                  pin_src, N_CELL)
    norm_src = jnp.where(deg_src > 0, deg_src ** -0.5, 0.0)
    feat = node_feat * norm_src[:, None]

    n_net_pad = 10112
    k_chunks = -(-E_PIN // (16 * 128))
    e_pad = 16 * k_chunks * 128
    src_p = jnp.concatenate(
        [pin_src, jnp.zeros((e_pad - E_PIN,), jnp.int32)]).reshape(16, k_chunks, 128)
    dst_p = jnp.concatenate(
        [pin_dst, jnp.full((e_pad - E_PIN,), n_net_pad - 1, jnp.int32)]
    ).reshape(16, k_chunks, 128)
    zc = jnp.zeros((N_CELL, _DAUG - 65), jnp.float32)
    table = jnp.concatenate([
        jnp.concatenate([feat[:, :64], jnp.ones((N_CELL, 1), jnp.float32), zc], 1),
        jnp.concatenate([feat[:, 64:], jnp.zeros((N_CELL, 1), jnp.float32), zc], 1),
    ], axis=0)
    parts = _make_segsum(n_net_pad, k_chunks, N_CELL)(table, src_p, dst_p)
    agg = jnp.concatenate([parts[0, :N_NET, :64], parts[1, :N_NET, :64]], axis=1)
    deg_dst = parts[0, :N_NET, 64]
    norm_dst = jnp.where(deg_dst > 0, deg_dst ** -0.5, 0.0)

    cell_out = _mm3(node_feat, neigh, neigh2,
                    t_self_W + g_self_W, t_neigh_W, g_neigh_W,
                    t_self_b + g_self_b, 1000)

    aggn = agg * norm_dst[:, None]
    net_out = _mm3(aggn, net_feat, jnp.zeros_like(net_feat),
                   gc_W, net_lin_W, jnp.zeros_like(net_lin_W),
                   gc_b + net_lin_b, 1000)
    return (cell_out, net_out)


# speedup vs baseline: 1.4119x; 1.2102x over previous
import functools

import jax
import jax.numpy as jnp
from jax import lax
from jax.experimental import pallas as pl
from jax.experimental.pallas import tpu as pltpu
from jax.experimental.pallas import tpu_sc as plsc


def _mm3_body(a, b, c, wa, wb, wc, bias, o):
    acc = jnp.dot(a[...], wa[...], preferred_element_type=jnp.float32)
    acc += jnp.dot(b[...], wb[...], preferred_element_type=jnp.float32)
    acc += jnp.dot(c[...], wc[...], preferred_element_type=jnp.float32)
    o[...] = acc + bias[...]


def _mm3(a, b, c, wa, wb, wc, bias, block_rows):
    n, d = a.shape
    do = wa.shape[1]
    row_spec = pl.BlockSpec((block_rows, d), lambda i: (i, 0))
    w_spec = pl.BlockSpec((d, do), lambda i: (0, 0))
    return pl.pallas_call(
        _mm3_body,
        grid=(n // block_rows,),
        in_specs=[row_spec, row_spec, row_spec, w_spec, w_spec, w_spec,
                  pl.BlockSpec((1, do), lambda i: (0, 0))],
        out_specs=pl.BlockSpec((block_rows, do), lambda i: (i, 0)),
        out_shape=jax.ShapeDtypeStruct((n, do), jnp.float32),
    )(a, b, c, wa, wb, wc, bias.reshape(1, do))


def _relu_mm_body(x, w, bias, o):
    o[...] = jax.nn.relu(
        jnp.dot(x[...], w[...], preferred_element_type=jnp.float32) + bias[...])


def _relu_mm(x, w, bias, block_rows):
    n, d = x.shape
    do = w.shape[1]
    return pl.pallas_call(
        _relu_mm_body,
        grid=(n // block_rows,),
        in_specs=[pl.BlockSpec((block_rows, d), lambda i: (i, 0)),
                  pl.BlockSpec((d, do), lambda i: (0, 0)),
                  pl.BlockSpec((1, do), lambda i: (0, 0))],
        out_specs=pl.BlockSpec((block_rows, do), lambda i: (i, 0)),
        out_shape=jax.ShapeDtypeStruct((n, do), jnp.float32),
    )(x, w, bias.reshape(1, do))


_NW = 32
_DAUG = 80


def _make_segsum(n_rows_padded, k_chunks, n_table):
    mesh = plsc.VectorSubcoreMesh(core_axis_name="c", subcore_axis_name="s")
    rows_per_sub = n_rows_padded // 16

    @functools.partial(
        pl.kernel, mesh=mesh,
        compiler_params=pltpu.CompilerParams(use_tc_tiling_on_sc=False),
        out_type=jax.ShapeDtypeStruct((2, n_rows_padded, _DAUG), jnp.float32),
        scratch_types=[
            pltpu.VMEM((k_chunks, 128), jnp.int32),
            pltpu.VMEM((k_chunks, 128), jnp.int32),
            pltpu.VMEM((2, 128, _DAUG), jnp.float32),
            pltpu.VMEM_SHARED((n_rows_padded, _DAUG), jnp.float32),
            pltpu.SemaphoreType.DMA,
            pltpu.SemaphoreType.DMA,
        ],
    )
    def seg_sum(table_hbm, src_hbm, dst_hbm, out_hbm,
                src_v, dst_v, rows_v, acc_sh, gsem, ssem):
        cid = lax.axis_index("c")
        sid = lax.axis_index("s")

        zrow = jnp.zeros((16,), jnp.float32)

        def zero_body(i, _):
            for q in range(_DAUG // 16):
                rows_v[0, i, pl.ds(q * 16, 16)] = zrow
            return 0

        lax.fori_loop(0, 128, zero_body, 0)
        base = sid * rows_per_sub
        nfull = rows_per_sub // 128
        for z in range(nfull):
            pltpu.sync_copy(rows_v.at[0],
                            acc_sh.at[pl.ds(base + z * 128, 128)])
        rem = rows_per_sub - nfull * 128
        if rem:
            pltpu.sync_copy(rows_v.at[0, pl.ds(0, rem)],
                            acc_sh.at[pl.ds(base + nfull * 128, rem)])
        plsc.subcore_barrier()

        pltpu.sync_copy(src_hbm.at[sid], src_v)
        pltpu.sync_copy(dst_hbm.at[sid], dst_v)
        off = (cid * n_table).astype(jnp.int32)

        def bump_body(i, _):
            r = lax.div(i, jnp.int32(8))
            q = lax.rem(i, jnp.int32(8))
            src_v[r, pl.ds(q * 16, 16)] = src_v[r, pl.ds(q * 16, 16)] + off
            return 0

        lax.fori_loop(0, k_chunks * 8, bump_body, 0)

        def chunk_body(j, _):
            slot = lax.rem(j, 2)
            pltpu.async_copy(table_hbm.at[src_v.at[j]], rows_v.at[slot],
                             gsem).wait()
            pltpu.sync_copy(rows_v.at[slot], acc_sh.at[dst_v.at[j]],
                            add=True)
            return 0

        lax.fori_loop(0, k_chunks, chunk_body, 0)
        plsc.subcore_barrier()

        for z in range(nfull):
            pltpu.sync_copy(acc_sh.at[pl.ds(base + z * 128, 128)],
                            rows_v.at[1])
            pltpu.sync_copy(rows_v.at[1],
                            out_hbm.at[cid, pl.ds(base + z * 128, 128)])
        if rem:
            pltpu.sync_copy(acc_sh.at[pl.ds(base + nfull * 128, rem)],
                            rows_v.at[1, pl.ds(0, rem)])
            pltpu.sync_copy(rows_v.at[1, pl.ds(0, rem)],
                            out_hbm.at[cid, pl.ds(base + nfull * 128, rem)])

    return seg_sum


_R = 1568
_CS = 2048
_CAP = 3072
_GB = 64


def _make_segmax(n_table, n_chunks):
    mesh = plsc.VectorSubcoreMesh(core_axis_name="c", subcore_axis_name="s")

    gdn = lax.GatherDimensionNumbers(
        offset_dims=(), collapsed_slice_dims=(0,), start_index_map=(0,))

    def _dg(v, idx):
        return lax.gather(v, idx[:, None], gdn, (1,),
                          mode=lax.GatherScatterMode.PROMISE_IN_BOUNDS)

    @functools.partial(
        pl.kernel, mesh=mesh,
        compiler_params=pltpu.CompilerParams(use_tc_tiling_on_sc=False,
                                             needs_layout_passes=False),
        out_type=jax.ShapeDtypeStruct((32 * _R, 64), jnp.int32),
        scratch_types=[
            pltpu.VMEM((_R, 64), jnp.int32),
            pltpu.VMEM((2, _CS), jnp.int32),
            pltpu.VMEM((2, _CS), jnp.int32),
            pltpu.VMEM((2, _CS), jnp.float32),
            pltpu.VMEM((_CAP + 16,), jnp.int32),
            pltpu.VMEM((_CAP + 16,), jnp.int32),
            pltpu.VMEM((_CAP + 16,), jnp.float32),
            pltpu.VMEM((16,), jnp.int32),
            pltpu.VMEM((2, _GB, 64), jnp.int32),
            pltpu.SMEM((2,), jnp.int32),
            pltpu.SemaphoreType.DMA,
            pltpu.SemaphoreType.DMA,
            pltpu.SemaphoreType.DMA,
            pltpu.SemaphoreType.DMA,
        ],
    )
    def seg_max(table_hbm, key_hbm, gidx_hbm, ew_hbm, acc_out,
                acc_v, keyb, gixb, ewb, glist, dlist, ewlist,
                offr, rows_v, cnts, csem0, csem1, gsem0, gsem1):
        cid = lax.axis_index("c")
        sid = lax.axis_index("s")
        wid = cid * 16 + sid
        base = wid * _R

        def init_body(i, _):
            neg1 = jnp.full((16,), -1082081408, jnp.int32)
            for q in range(4):
                acc_v[i, pl.ds(q * 16, 16)] = neg1
            return 0

        lax.fori_loop(0, _R, init_body, 0)

        def zlist_body(i, _):
            glist[pl.ds(i * 16, 16)] = jnp.zeros((16,), jnp.int32)
            return 0

        lax.fori_loop(0, (_CAP + 16) // 16, zlist_body, 0)
        offr[...] = jnp.zeros((16,), jnp.int32)
        cnts[0] = 0
        cnts[1] = 0

        def drain(n_matched):
            nb = lax.div(n_matched + (_GB - 1), jnp.int32(_GB))

            def fire_gather(b, slot):
                sem = [gsem0, gsem1][slot]
                pltpu.async_copy(
                    table_hbm.at[glist.at[pl.ds(b * _GB, _GB)]],
                    rows_v.at[slot], sem)

            @pl.when(nb > 0)
            def _():
                fire_gather(0, 0)

            def batch_body(b, _):
                slot = lax.rem(b, 2)

                for sl in range(2):
                    @pl.when((b + 1 < nb) & (slot == sl))
                    def _(sl=sl):
                        fire_gather(b + 1, 1 - sl)

                for sl in range(2):
                    @pl.when(slot == sl)
                    def _(sl=sl):
                        pltpu.make_async_copy(
                            table_hbm.at[glist.at[pl.ds(b * _GB, _GB)]],
                            rows_v.at[sl], [gsem0, gsem1][sl]).wait()
                for g in range(_GB // 16):
                    j0 = b * _GB + g * 16

                    @pl.when(j0 < n_matched)
                    def _(j0=j0, g=g, slot=slot):
                        dvec = dlist[pl.ds(j0, 16)]
                        evec = ewlist[pl.ds(j0, 16)]
                        for l in range(16):

                            @pl.when(j0 + l < n_matched)
                            def _(l=l, dvec=dvec, evec=evec, g=g, slot=slot):
                                dloc = dvec[l]
                                s32 = jnp.broadcast_to(evec[l], (16,))
                                scale = plsc.pack(
                                    s32, s32,
                                    format=plsc.PackFormat.INTERLEAVED)
                                jl = g * 16 + l
                                for q in range(4):
                                    a = acc_v[dloc, pl.ds(q * 16, 16)]
                                    mi = rows_v[slot, jl, pl.ds(q * 16, 16)]
                                    ab = plsc.bitcast(a, jnp.bfloat16)
                                    mb = plsc.bitcast(mi, jnp.bfloat16)
                                    nv = jnp.maximum(ab, mb * scale)
                                    acc_v[dloc, pl.ds(q * 16, 16)] = (
                                        plsc.bitcast(nv, jnp.int32))
                return 0

            lax.fori_loop(0, nb, batch_body, 0)

        def fire_chunk(c, slot):
            sem = [csem0, csem1][slot]
            pltpu.async_copy(key_hbm.at[pl.ds(c * _CS, _CS)],
                             keyb.at[slot], sem)
            pltpu.async_copy(gidx_hbm.at[pl.ds(c * _CS, _CS)],
                             gixb.at[slot], sem)
            pltpu.async_copy(ew_hbm.at[pl.ds(c * _CS, _CS)],
                             ewb.at[slot], sem)

        fire_chunk(0, 0)

        def chunk_body(c, _):
            cslot = lax.rem(c, 2)

            for sl in range(2):
                @pl.when((c + 1 < n_chunks) & (cslot == sl))
                def _(sl=sl):
                    fire_chunk(c + 1, 1 - sl)

            for sl in range(2):
                @pl.when(cslot == sl)
                def _(sl=sl):
                    sem = [csem0, csem1][sl]
                    pltpu.make_async_copy(
                        key_hbm.at[pl.ds(c * _CS, _CS)], keyb.at[sl],
                        sem).wait()
                    pltpu.make_async_copy(
                        gidx_hbm.at[pl.ds(c * _CS, _CS)], gixb.at[sl],
                        sem).wait()
                    pltpu.make_async_copy(
                        ew_hbm.at[pl.ds(c * _CS, _CS)], ewb.at[sl],
                        sem).wait()

            def vreg_body(i, _):
                offv = offr[...]
                k = keyb[cslot, pl.ds(i * 16, 16)]
                local = k - base
                inb = (local >= 0) & (local < _R)
                ib = jnp.where(inb, jnp.ones((16,), jnp.int32),
                               jnp.zeros((16,), jnp.int32))
                pos = offv + plsc.cumsum(ib) - 1
                plsc.store_scatter(glist, [pos],
                                   gixb[cslot, pl.ds(i * 16, 16)], mask=inb)
                plsc.store_scatter(dlist, [pos], local, mask=inb)
                plsc.store_scatter(ewlist, [pos],
                                   ewb[cslot, pl.ds(i * 16, 16)], mask=inb)
                offr[...] = offv + plsc.all_reduce_population_count(inb)
                return 0

            lax.fori_loop(0, _CS // 16, vreg_body, 0)
            n_matched = offr[...][0]
            do_drain = n_matched >= (_CAP - _CS)

            @pl.when(do_drain)
            def _():
                drain(n_matched)
                offr[...] = jnp.zeros((16,), jnp.int32)

            return 0

        lax.fori_loop(0, n_chunks, chunk_body, 0)
        drain(offr[...][0])

        pltpu.sync_copy(acc_v, acc_out.at[pl.ds(base, _R)])

    return seg_max


def _pack_bf16(x):
    b = x.astype(jnp.bfloat16)
    return jax.lax.bitcast_convert_type(
        b.reshape(x.shape[0], x.shape[1] // 2, 2), jnp.int32)


def _unpack_bf16(p):
    b = jax.lax.bitcast_convert_type(p, jnp.bfloat16)
    return b.reshape(p.shape[0], p.shape[1] * 2)


def _segmax(table_f32, key, gidx, ew, n_out):
    e = key.shape[0]
    n_chunks = -(-e // _CS)
    e_pad = n_chunks * _CS
    pad = e_pad - e
    key_p = jnp.concatenate([key, jnp.full((pad,), 2**30, jnp.int32)])
    gidx_p = jnp.concatenate([gidx, jnp.zeros((pad,), jnp.int32)])
    ew_p = jnp.concatenate([ew, jnp.zeros((pad,), jnp.float32)])
    table = _pack_bf16(table_f32)
    acc = _make_segmax(table.shape[0], n_chunks)(table, key_p, gidx_p, ew_p)
    return jax.nn.relu(_unpack_bf16(acc)[:n_out].astype(jnp.float32))


def kernel(node_feat, net_feat, pin_feat, edge_feat, gc_W, gc_b, t_pool_W, t_pool_b,
           t_neigh_W, t_self_W, t_self_b, g_pool_W, g_pool_b, g_neigh_W, g_self_W,
           g_self_b, topo_w_W, topo_w_b, geom_w_W, geom_w_b, net_lin_W, net_lin_b,
           pin_src, pin_dst, near_src, near_dst):
    N_CELL, _ = node_feat.shape
    N_NET, _ = net_feat.shape
    E_PIN = pin_src.shape[0]
    E_NEAR = near_src.shape[0]

    ew_pin = jax.nn.sigmoid(pin_feat @ topo_w_W + topo_w_b)
    ew_near = jax.nn.sigmoid(edge_feat @ geom_w_W + geom_w_b)

    h_pool = _relu_mm(net_feat, t_pool_W, t_pool_b, 1000)
    neigh = _segmax(h_pool, pin_src, pin_dst, ew_pin[:, 0], N_CELL)

    h_pool2 = _relu_mm(node_feat, g_pool_W, g_pool_b, 1000)
    neigh2 = _segmax(h_pool2, near_dst, near_src, ew_near[:, 0], N_CELL)

    deg_src = jax.ops.segment_sum(jnp.ones((E_PIN,), jnp.float32),
                                  pin_src, N_CELL)
    norm_src = jnp.where(deg_src > 0, deg_src ** -0.5, 0.0)
    feat = node_feat * norm_src[:, None]

    n_net_pad = 10112
    k_chunks = -(-E_PIN // (16 * 128))
    e_pad = 16 * k_chunks * 128
    src_p = jnp.concatenate(
        [pin_src, jnp.zeros((e_pad - E_PIN,), jnp.int32)]).reshape(16, k_chunks, 128)
    dst_p = jnp.concatenate(
        [pin_dst, jnp.full((e_pad - E_PIN,), n_net_pad - 1, jnp.int32)]
    ).reshape(16, k_chunks, 128)
    zc = jnp.zeros((N_CELL, _DAUG - 65), jnp.float32)
    table = jnp.concatenate([
        jnp.concatenate([feat[:, :64], jnp.ones((N_CELL, 1), jnp.float32), zc], 1),
        jnp.concatenate([feat[:, 64:], jnp.zeros((N_CELL, 1), jnp.float32), zc], 1),
    ], axis=0)
    parts = _make_segsum(n_net_pad, k_chunks, N_CELL)(table, src_p, dst_p)
    agg = jnp.concatenate([parts[0, :N_NET, :64], parts[1, :N_NET, :64]], axis=1)
    deg_dst = parts[0, :N_NET, 64]
    norm_dst = jnp.where(deg_dst > 0, deg_dst ** -0.5, 0.0)

    cell_out = _mm3(node_feat, neigh, neigh2,
                    t_self_W + g_self_W, t_neigh_W, g_neigh_W,
                    t_self_b + g_self_b, 1000)

    aggn = agg * norm_dst[:, None]
    net_out = _mm3(aggn, net_feat, jnp.zeros_like(net_feat),
                   gc_W, net_lin_W, jnp.zeros_like(net_lin_W),
                   gc_b + net_lin_b, 1000)
    return (cell_out, net_out)


# speedup vs baseline: 1.5607x; 1.1054x over previous
import functools

import jax
import jax.numpy as jnp
from jax import lax
from jax.experimental import pallas as pl
from jax.experimental.pallas import tpu as pltpu
from jax.experimental.pallas import tpu_sc as plsc


def _mm3_body(a, b, c, wa, wb, wc, bias, o):
    acc = jnp.dot(a[...], wa[...], preferred_element_type=jnp.float32)
    acc += jnp.dot(b[...], wb[...], preferred_element_type=jnp.float32)
    acc += jnp.dot(c[...], wc[...], preferred_element_type=jnp.float32)
    o[...] = acc + bias[...]


def _mm3(a, b, c, wa, wb, wc, bias, block_rows):
    n, d = a.shape
    do = wa.shape[1]
    row_spec = pl.BlockSpec((block_rows, d), lambda i: (i, 0))
    w_spec = pl.BlockSpec((d, do), lambda i: (0, 0))
    return pl.pallas_call(
        _mm3_body,
        grid=(n // block_rows,),
        in_specs=[row_spec, row_spec, row_spec, w_spec, w_spec, w_spec,
                  pl.BlockSpec((1, do), lambda i: (0, 0))],
        out_specs=pl.BlockSpec((block_rows, do), lambda i: (i, 0)),
        out_shape=jax.ShapeDtypeStruct((n, do), jnp.float32),
    )(a, b, c, wa, wb, wc, bias.reshape(1, do))


def _relu_mm_body(x, w, bias, o):
    o[...] = jax.nn.relu(
        jnp.dot(x[...], w[...], preferred_element_type=jnp.float32) + bias[...])


def _relu_mm(x, w, bias, block_rows):
    n, d = x.shape
    do = w.shape[1]
    return pl.pallas_call(
        _relu_mm_body,
        grid=(n // block_rows,),
        in_specs=[pl.BlockSpec((block_rows, d), lambda i: (i, 0)),
                  pl.BlockSpec((d, do), lambda i: (0, 0)),
                  pl.BlockSpec((1, do), lambda i: (0, 0))],
        out_specs=pl.BlockSpec((block_rows, do), lambda i: (i, 0)),
        out_shape=jax.ShapeDtypeStruct((n, do), jnp.float32),
    )(x, w, bias.reshape(1, do))


_NW = 32
_DAUG = 80


def _make_segsum(n_rows_padded, k_chunks, n_table):
    mesh = plsc.VectorSubcoreMesh(core_axis_name="c", subcore_axis_name="s")
    rows_per_sub = n_rows_padded // 16

    @functools.partial(
        pl.kernel, mesh=mesh,
        compiler_params=pltpu.CompilerParams(use_tc_tiling_on_sc=False),
        out_type=jax.ShapeDtypeStruct((2, n_rows_padded, _DAUG), jnp.float32),
        scratch_types=[
            pltpu.VMEM((k_chunks, 128), jnp.int32),
            pltpu.VMEM((k_chunks, 128), jnp.int32),
            pltpu.VMEM((2, 128, _DAUG), jnp.float32),
            pltpu.VMEM_SHARED((n_rows_padded, _DAUG), jnp.float32),
            pltpu.SemaphoreType.DMA,
            pltpu.SemaphoreType.DMA,
        ],
    )
    def seg_sum(table_hbm, src_hbm, dst_hbm, out_hbm,
                src_v, dst_v, rows_v, acc_sh, gsem, ssem):
        cid = lax.axis_index("c")
        sid = lax.axis_index("s")

        zrow = jnp.zeros((16,), jnp.float32)

        def zero_body(i, _):
            for q in range(_DAUG // 16):
                rows_v[0, i, pl.ds(q * 16, 16)] = zrow
            return 0

        lax.fori_loop(0, 128, zero_body, 0)
        base = sid * rows_per_sub
        nfull = rows_per_sub // 128
        for z in range(nfull):
            pltpu.sync_copy(rows_v.at[0],
                            acc_sh.at[pl.ds(base + z * 128, 128)])
        rem = rows_per_sub - nfull * 128
        if rem:
            pltpu.sync_copy(rows_v.at[0, pl.ds(0, rem)],
                            acc_sh.at[pl.ds(base + nfull * 128, rem)])
        plsc.subcore_barrier()

        pltpu.sync_copy(src_hbm.at[sid], src_v)
        pltpu.sync_copy(dst_hbm.at[sid], dst_v)
        off = (cid * n_table).astype(jnp.int32)

        def bump_body(i, _):
            r = lax.div(i, jnp.int32(8))
            q = lax.rem(i, jnp.int32(8))
            src_v[r, pl.ds(q * 16, 16)] = src_v[r, pl.ds(q * 16, 16)] + off
            return 0

        lax.fori_loop(0, k_chunks * 8, bump_body, 0)

        def chunk_body(j, _):
            slot = lax.rem(j, 2)
            pltpu.async_copy(table_hbm.at[src_v.at[j]], rows_v.at[slot],
                             gsem).wait()
            pltpu.sync_copy(rows_v.at[slot], acc_sh.at[dst_v.at[j]],
                            add=True)
            return 0

        lax.fori_loop(0, k_chunks, chunk_body, 0)
        plsc.subcore_barrier()

        for z in range(nfull):
            pltpu.sync_copy(acc_sh.at[pl.ds(base + z * 128, 128)],
                            rows_v.at[1])
            pltpu.sync_copy(rows_v.at[1],
                            out_hbm.at[cid, pl.ds(base + z * 128, 128)])
        if rem:
            pltpu.sync_copy(acc_sh.at[pl.ds(base + nfull * 128, rem)],
                            rows_v.at[1, pl.ds(0, rem)])
            pltpu.sync_copy(rows_v.at[1, pl.ds(0, rem)],
                            out_hbm.at[cid, pl.ds(base + nfull * 128, rem)])

    return seg_sum


_R = 1568
_CS = 2048
_CAP = 3072
_GB = 64


def _make_segmax(n_table, n_chunks):
    mesh = plsc.VectorSubcoreMesh(core_axis_name="c", subcore_axis_name="s")

    gdn = lax.GatherDimensionNumbers(
        offset_dims=(), collapsed_slice_dims=(0,), start_index_map=(0,))

    def _dg(v, idx):
        return lax.gather(v, idx[:, None], gdn, (1,),
                          mode=lax.GatherScatterMode.PROMISE_IN_BOUNDS)

    @functools.partial(
        pl.kernel, mesh=mesh,
        compiler_params=pltpu.CompilerParams(use_tc_tiling_on_sc=False,
                                             needs_layout_passes=False),
        out_type=jax.ShapeDtypeStruct((32 * _R, 64), jnp.int32),
        scratch_types=[
            pltpu.VMEM((_R, 64), jnp.int32),
            pltpu.VMEM((2, _CS), jnp.int32),
            pltpu.VMEM((2, _CS), jnp.int32),
            pltpu.VMEM((2, _CS), jnp.float32),
            pltpu.VMEM((_CAP + 16,), jnp.int32),
            pltpu.VMEM((_CAP + 16,), jnp.int32),
            pltpu.VMEM((_CAP + 16,), jnp.float32),
            pltpu.VMEM((16,), jnp.int32),
            pltpu.VMEM((2, _GB, 64), jnp.int32),
            pltpu.SMEM((2,), jnp.int32),
            pltpu.SemaphoreType.DMA,
            pltpu.SemaphoreType.DMA,
            pltpu.SemaphoreType.DMA,
            pltpu.SemaphoreType.DMA,
        ],
    )
    def seg_max(table_hbm, key_hbm, gidx_hbm, ew_hbm, acc_out,
                acc_v, keyb, gixb, ewb, glist, dlist, ewlist,
                offr, rows_v, cnts, csem0, csem1, gsem0, gsem1):
        cid = lax.axis_index("c")
        sid = lax.axis_index("s")
        wid = cid * 16 + sid
        base = wid * _R

        def init_body(i, _):
            neg1 = jnp.full((16,), -1082081408, jnp.int32)
            for q in range(4):
                acc_v[i, pl.ds(q * 16, 16)] = neg1
            return 0

        lax.fori_loop(0, _R, init_body, 0)

        def zlist_body(i, _):
            glist[pl.ds(i * 16, 16)] = jnp.zeros((16,), jnp.int32)
            return 0

        lax.fori_loop(0, (_CAP + 16) // 16, zlist_body, 0)
        offr[...] = jnp.zeros((16,), jnp.int32)
        cnts[0] = 0
        cnts[1] = 0

        def drain(n_matched):
            nb = lax.div(n_matched + (_GB - 1), jnp.int32(_GB))

            def fire_gather(b, slot):
                sem = [gsem0, gsem1][slot]
                pltpu.async_copy(
                    table_hbm.at[glist.at[pl.ds(b * _GB, _GB)]],
                    rows_v.at[slot], sem)

            @pl.when(nb > 0)
            def _():
                fire_gather(0, 0)

            def batch_body(b, _):
                slot = lax.rem(b, 2)

                for sl in range(2):
                    @pl.when((b + 1 < nb) & (slot == sl))
                    def _(sl=sl):
                        fire_gather(b + 1, 1 - sl)

                for sl in range(2):
                    @pl.when(slot == sl)
                    def _(sl=sl):
                        pltpu.make_async_copy(
                            table_hbm.at[glist.at[pl.ds(b * _GB, _GB)]],
                            rows_v.at[sl], [gsem0, gsem1][sl]).wait()
                for g in range(_GB // 16):
                    j0 = b * _GB + g * 16

                    @pl.when(j0 < n_matched)
                    def _(j0=j0, g=g, slot=slot):
                        dvec = dlist[pl.ds(j0, 16)]
                        evec = ewlist[pl.ds(j0, 16)]
                        for l in range(16):

                            @pl.when(j0 + l < n_matched)
                            def _(l=l, dvec=dvec, evec=evec, g=g, slot=slot):
                                dloc = dvec[l]
                                s32 = jnp.broadcast_to(evec[l], (16,))
                                scale = plsc.pack(
                                    s32, s32,
                                    format=plsc.PackFormat.INTERLEAVED)
                                jl = g * 16 + l
                                for q in range(4):
                                    a = acc_v[dloc, pl.ds(q * 16, 16)]
                                    mi = rows_v[slot, jl, pl.ds(q * 16, 16)]
                                    ab = plsc.bitcast(a, jnp.bfloat16)
                                    mb = plsc.bitcast(mi, jnp.bfloat16)
                                    nv = jnp.maximum(ab, mb * scale)
                                    acc_v[dloc, pl.ds(q * 16, 16)] = (
                                        plsc.bitcast(nv, jnp.int32))
                return 0

            lax.fori_loop(0, nb, batch_body, 0)

        def fire_chunk(c, slot):
            sem = [csem0, csem1][slot]
            pltpu.async_copy(key_hbm.at[pl.ds(c * _CS, _CS)],
                             keyb.at[slot], sem)
            pltpu.async_copy(gidx_hbm.at[pl.ds(c * _CS, _CS)],
                             gixb.at[slot], sem)
            pltpu.async_copy(ew_hbm.at[pl.ds(c * _CS, _CS)],
                             ewb.at[slot], sem)

        fire_chunk(0, 0)

        def chunk_body(c, _):
            cslot = lax.rem(c, 2)

            for sl in range(2):
                @pl.when((c + 1 < n_chunks) & (cslot == sl))
                def _(sl=sl):
                    fire_chunk(c + 1, 1 - sl)

            for sl in range(2):
                @pl.when(cslot == sl)
                def _(sl=sl):
                    sem = [csem0, csem1][sl]
                    pltpu.make_async_copy(
                        key_hbm.at[pl.ds(c * _CS, _CS)], keyb.at[sl],
                        sem).wait()
                    pltpu.make_async_copy(
                        gidx_hbm.at[pl.ds(c * _CS, _CS)], gixb.at[sl],
                        sem).wait()
                    pltpu.make_async_copy(
                        ew_hbm.at[pl.ds(c * _CS, _CS)], ewb.at[sl],
                        sem).wait()

            def vreg_body(i, _):
                offv = offr[...]
                ks = [keyb[cslot, pl.ds((i * 4 + u) * 16, 16)]
                      for u in range(4)]
                locs = [k - base for k in ks]
                inbs = [(lo >= 0) & (lo < _R) for lo in locs]
                one = jnp.ones((16,), jnp.int32)
                zero = jnp.zeros((16,), jnp.int32)
                ibs = [jnp.where(m, one, zero) for m in inbs]
                cums = [plsc.cumsum(b) for b in ibs]
                pcs = [plsc.all_reduce_population_count(m) for m in inbs]
                for u in range(4):
                    pos = offv + cums[u] - 1
                    plsc.store_scatter(
                        glist, [pos],
                        gixb[cslot, pl.ds((i * 4 + u) * 16, 16)],
                        mask=inbs[u])
                    plsc.store_scatter(dlist, [pos], locs[u], mask=inbs[u])
                    plsc.store_scatter(
                        ewlist, [pos],
                        ewb[cslot, pl.ds((i * 4 + u) * 16, 16)],
                        mask=inbs[u])
                    offv = offv + pcs[u]
                offr[...] = offv
                return 0

            lax.fori_loop(0, _CS // 64, vreg_body, 0)
            n_matched = offr[...][0]
            do_drain = n_matched >= (_CAP - _CS)

            @pl.when(do_drain)
            def _():
                drain(n_matched)
                offr[...] = jnp.zeros((16,), jnp.int32)

            return 0

        lax.fori_loop(0, n_chunks, chunk_body, 0)
        drain(offr[...][0])

        pltpu.sync_copy(acc_v, acc_out.at[pl.ds(base, _R)])

    return seg_max


def _pack_bf16(x):
    b = x.astype(jnp.bfloat16)
    return jax.lax.bitcast_convert_type(
        b.reshape(x.shape[0], x.shape[1] // 2, 2), jnp.int32)


def _unpack_bf16(p):
    b = jax.lax.bitcast_convert_type(p, jnp.bfloat16)
    return b.reshape(p.shape[0], p.shape[1] * 2)


def _segmax(table_f32, key, gidx, ew, n_out):
    e = key.shape[0]
    n_chunks = -(-e // _CS)
    e_pad = n_chunks * _CS
    pad = e_pad - e
    key_p = jnp.concatenate([key, jnp.full((pad,), 2**30, jnp.int32)])
    gidx_p = jnp.concatenate([gidx, jnp.zeros((pad,), jnp.int32)])
    ew_p = jnp.concatenate([ew, jnp.zeros((pad,), jnp.float32)])
    table = _pack_bf16(table_f32)
    acc = _make_segmax(table.shape[0], n_chunks)(table, key_p, gidx_p, ew_p)
    return jax.nn.relu(_unpack_bf16(acc)[:n_out].astype(jnp.float32))


def kernel(node_feat, net_feat, pin_feat, edge_feat, gc_W, gc_b, t_pool_W, t_pool_b,
           t_neigh_W, t_self_W, t_self_b, g_pool_W, g_pool_b, g_neigh_W, g_self_W,
           g_self_b, topo_w_W, topo_w_b, geom_w_W, geom_w_b, net_lin_W, net_lin_b,
           pin_src, pin_dst, near_src, near_dst):
    N_CELL, _ = node_feat.shape
    N_NET, _ = net_feat.shape
    E_PIN = pin_src.shape[0]
    E_NEAR = near_src.shape[0]

    ew_pin = jax.nn.sigmoid(pin_feat @ topo_w_W + topo_w_b)
    ew_near = jax.nn.sigmoid(edge_feat @ geom_w_W + geom_w_b)

    h_pool = _relu_mm(net_feat, t_pool_W, t_pool_b, 1000)
    neigh = _segmax(h_pool, pin_src, pin_dst, ew_pin[:, 0], N_CELL)

    h_pool2 = _relu_mm(node_feat, g_pool_W, g_pool_b, 1000)
    neigh2 = _segmax(h_pool2, near_dst, near_src, ew_near[:, 0], N_CELL)

    deg_src = jax.ops.segment_sum(jnp.ones((E_PIN,), jnp.float32),
                                  pin_src, N_CELL)
    norm_src = jnp.where(deg_src > 0, deg_src ** -0.5, 0.0)
    feat = node_feat * norm_src[:, None]

    n_net_pad = 10112
    k_chunks = -(-E_PIN // (16 * 128))
    e_pad = 16 * k_chunks * 128
    src_p = jnp.concatenate(
        [pin_src, jnp.zeros((e_pad - E_PIN,), jnp.int32)]).reshape(16, k_chunks, 128)
    dst_p = jnp.concatenate(
        [pin_dst, jnp.full((e_pad - E_PIN,), n_net_pad - 1, jnp.int32)]
    ).reshape(16, k_chunks, 128)
    zc = jnp.zeros((N_CELL, _DAUG - 65), jnp.float32)
    table = jnp.concatenate([
        jnp.concatenate([feat[:, :64], jnp.ones((N_CELL, 1), jnp.float32), zc], 1),
        jnp.concatenate([feat[:, 64:], jnp.zeros((N_CELL, 1), jnp.float32), zc], 1),
    ], axis=0)
    parts = _make_segsum(n_net_pad, k_chunks, N_CELL)(table, src_p, dst_p)
    agg = jnp.concatenate([parts[0, :N_NET, :64], parts[1, :N_NET, :64]], axis=1)
    deg_dst = parts[0, :N_NET, 64]
    norm_dst = jnp.where(deg_dst > 0, deg_dst ** -0.5, 0.0)

    cell_out = _mm3(node_feat, neigh, neigh2,
                    t_self_W + g_self_W, t_neigh_W, g_neigh_W,
                    t_self_b + g_self_b, 1000)

    aggn = agg * norm_dst[:, None]
    net_out = _mm3(aggn, net_feat, jnp.zeros_like(net_feat),
                   gc_W, net_lin_W, jnp.zeros_like(net_lin_W),
                   gc_b + net_lin_b, 1000)
    return (cell_out, net_out)


# speedup vs baseline: 2.0393x; 1.3066x over previous
import functools

import jax
import jax.numpy as jnp
from jax import lax
from jax.experimental import pallas as pl
from jax.experimental.pallas import tpu as pltpu
from jax.experimental.pallas import tpu_sc as plsc


def _mm3_body(a, b, c, wa, wb, wc, bias, o):
    acc = jnp.dot(a[...], wa[...], preferred_element_type=jnp.float32)
    acc += jnp.dot(b[...], wb[...], preferred_element_type=jnp.float32)
    acc += jnp.dot(c[...], wc[...], preferred_element_type=jnp.float32)
    o[...] = acc + bias[...]


def _mm3(a, b, c, wa, wb, wc, bias, block_rows):
    n, d = a.shape
    do = wa.shape[1]
    row_spec = pl.BlockSpec((block_rows, d), lambda i: (i, 0))
    w_spec = pl.BlockSpec((d, do), lambda i: (0, 0))
    return pl.pallas_call(
        _mm3_body,
        grid=(n // block_rows,),
        in_specs=[row_spec, row_spec, row_spec, w_spec, w_spec, w_spec,
                  pl.BlockSpec((1, do), lambda i: (0, 0))],
        out_specs=pl.BlockSpec((block_rows, do), lambda i: (i, 0)),
        out_shape=jax.ShapeDtypeStruct((n, do), jnp.float32),
    )(a, b, c, wa, wb, wc, bias.reshape(1, do))


def _relu_mm_body(x, w, bias, o):
    o[...] = jax.nn.relu(
        jnp.dot(x[...], w[...], preferred_element_type=jnp.float32) + bias[...])


def _relu_mm(x, w, bias, block_rows):
    n, d = x.shape
    do = w.shape[1]
    return pl.pallas_call(
        _relu_mm_body,
        grid=(n // block_rows,),
        in_specs=[pl.BlockSpec((block_rows, d), lambda i: (i, 0)),
                  pl.BlockSpec((d, do), lambda i: (0, 0)),
                  pl.BlockSpec((1, do), lambda i: (0, 0))],
        out_specs=pl.BlockSpec((block_rows, do), lambda i: (i, 0)),
        out_shape=jax.ShapeDtypeStruct((n, do), jnp.float32),
    )(x, w, bias.reshape(1, do))


_NW = 32
_DAUG = 80


def _make_segsum(n_rows_padded, k_chunks, n_table):
    mesh = plsc.VectorSubcoreMesh(core_axis_name="c", subcore_axis_name="s")
    rows_per_sub = n_rows_padded // 16

    @functools.partial(
        pl.kernel, mesh=mesh,
        compiler_params=pltpu.CompilerParams(use_tc_tiling_on_sc=False),
        out_type=jax.ShapeDtypeStruct((2, n_rows_padded, _DAUG), jnp.float32),
        scratch_types=[
            pltpu.VMEM((k_chunks, 128), jnp.int32),
            pltpu.VMEM((k_chunks, 128), jnp.int32),
            pltpu.VMEM((2, 128, _DAUG), jnp.float32),
            pltpu.VMEM_SHARED((n_rows_padded, _DAUG), jnp.float32),
            pltpu.SemaphoreType.DMA,
            pltpu.SemaphoreType.DMA,
        ],
    )
    def seg_sum(table_hbm, src_hbm, dst_hbm, out_hbm,
                src_v, dst_v, rows_v, acc_sh, gsem, ssem):
        cid = lax.axis_index("c")
        sid = lax.axis_index("s")

        zrow = jnp.zeros((16,), jnp.float32)

        def zero_body(i, _):
            for q in range(_DAUG // 16):
                rows_v[0, i, pl.ds(q * 16, 16)] = zrow
            return 0

        lax.fori_loop(0, 128, zero_body, 0)
        base = sid * rows_per_sub
        nfull = rows_per_sub // 128
        for z in range(nfull):
            pltpu.sync_copy(rows_v.at[0],
                            acc_sh.at[pl.ds(base + z * 128, 128)])
        rem = rows_per_sub - nfull * 128
        if rem:
            pltpu.sync_copy(rows_v.at[0, pl.ds(0, rem)],
                            acc_sh.at[pl.ds(base + nfull * 128, rem)])
        plsc.subcore_barrier()

        pltpu.sync_copy(src_hbm.at[sid], src_v)
        pltpu.sync_copy(dst_hbm.at[sid], dst_v)
        off = (cid * n_table).astype(jnp.int32)

        def bump_body(i, _):
            r = lax.div(i, jnp.int32(8))
            q = lax.rem(i, jnp.int32(8))
            src_v[r, pl.ds(q * 16, 16)] = src_v[r, pl.ds(q * 16, 16)] + off
            return 0

        lax.fori_loop(0, k_chunks * 8, bump_body, 0)

        def chunk_body(j, _):
            slot = lax.rem(j, 2)
            pltpu.async_copy(table_hbm.at[src_v.at[j]], rows_v.at[slot],
                             gsem).wait()
            pltpu.sync_copy(rows_v.at[slot], acc_sh.at[dst_v.at[j]],
                            add=True)
            return 0

        lax.fori_loop(0, k_chunks, chunk_body, 0)
        plsc.subcore_barrier()

        for z in range(nfull):
            pltpu.sync_copy(acc_sh.at[pl.ds(base + z * 128, 128)],
                            rows_v.at[1])
            pltpu.sync_copy(rows_v.at[1],
                            out_hbm.at[cid, pl.ds(base + z * 128, 128)])
        if rem:
            pltpu.sync_copy(acc_sh.at[pl.ds(base + nfull * 128, rem)],
                            rows_v.at[1, pl.ds(0, rem)])
            pltpu.sync_copy(rows_v.at[1, pl.ds(0, rem)],
                            out_hbm.at[cid, pl.ds(base + nfull * 128, rem)])

    return seg_sum


_R = 1568
_CS = 2048
_CAP = 2560
_GB = 48


def _make_segmax(n_table, n_chunks, with_counts=False):
    mesh = plsc.VectorSubcoreMesh(core_axis_name="c", subcore_axis_name="s")

    gdn = lax.GatherDimensionNumbers(
        offset_dims=(), collapsed_slice_dims=(0,), start_index_map=(0,))

    def _dg(v, idx):
        return lax.gather(v, idx[:, None], gdn, (1,),
                          mode=lax.GatherScatterMode.PROMISE_IN_BOUNDS)

    @functools.partial(
        pl.kernel, mesh=mesh,
        compiler_params=pltpu.CompilerParams(use_tc_tiling_on_sc=False,
                                             needs_layout_passes=False),
        out_type=((jax.ShapeDtypeStruct((32 * _R, 64), jnp.int32),
                   jax.ShapeDtypeStruct((32 * _R,), jnp.float32))
                  if with_counts else
                  jax.ShapeDtypeStruct((32 * _R, 64), jnp.int32)),
        scratch_types=[
            pltpu.VMEM((_R, 64), jnp.int32),
            pltpu.VMEM((_R,), jnp.float32),
            pltpu.VMEM((2, _CS), jnp.int32),
            pltpu.VMEM((2, _CS), jnp.int32),
            pltpu.VMEM((2, _CS), jnp.float32),
            pltpu.VMEM((_CAP + 16,), jnp.int32),
            pltpu.VMEM((_CAP + 16,), jnp.int32),
            pltpu.VMEM((_CAP + 16,), jnp.float32),
            pltpu.VMEM((16,), jnp.int32),
            pltpu.VMEM((2, _GB, 64), jnp.int32),
            pltpu.SMEM((2,), jnp.int32),
            pltpu.SemaphoreType.DMA,
            pltpu.SemaphoreType.DMA,
            pltpu.SemaphoreType.DMA,
            pltpu.SemaphoreType.DMA,
        ],
    )
    def seg_max(table_hbm, key_hbm, gidx_hbm, ew_hbm, *rest):
        if with_counts:
            (acc_out, cnt_out, acc_v, cnt_v, keyb, gixb, ewb, glist, dlist,
             ewlist, offr, rows_v, cnts, csem0, csem1, gsem0, gsem1) = rest
        else:
            (acc_out, acc_v, cnt_v, keyb, gixb, ewb, glist, dlist,
             ewlist, offr, rows_v, cnts, csem0, csem1, gsem0, gsem1) = rest
        cid = lax.axis_index("c")
        sid = lax.axis_index("s")
        wid = cid * 16 + sid
        base = wid * _R

        def init_body(i, _):
            neg1 = jnp.full((16,), -1082081408, jnp.int32)
            for q in range(4):
                acc_v[i, pl.ds(q * 16, 16)] = neg1
            return 0

        lax.fori_loop(0, _R, init_body, 0)

        if with_counts:
            def initc_body(i, _):
                cnt_v[pl.ds(i * 16, 16)] = jnp.zeros((16,), jnp.float32)
                return 0

            lax.fori_loop(0, _R // 16, initc_body, 0)

        def zlist_body(i, _):
            glist[pl.ds(i * 16, 16)] = jnp.zeros((16,), jnp.int32)
            return 0

        lax.fori_loop(0, (_CAP + 16) // 16, zlist_body, 0)
        offr[...] = jnp.zeros((16,), jnp.int32)
        cnts[0] = 0
        cnts[1] = 0

        def drain(n_matched):
            nb = lax.div(n_matched + (_GB - 1), jnp.int32(_GB))

            def fire_gather(b, slot):
                sem = [gsem0, gsem1][slot]
                pltpu.async_copy(
                    table_hbm.at[glist.at[pl.ds(b * _GB, _GB)]],
                    rows_v.at[slot], sem)

            @pl.when(nb > 0)
            def _():
                fire_gather(0, 0)

            def batch_body(b, _):
                slot = lax.rem(b, 2)

                for sl in range(2):
                    @pl.when((b + 1 < nb) & (slot == sl))
                    def _(sl=sl):
                        fire_gather(b + 1, 1 - sl)

                for sl in range(2):
                    @pl.when(slot == sl)
                    def _(sl=sl):
                        pltpu.make_async_copy(
                            table_hbm.at[glist.at[pl.ds(b * _GB, _GB)]],
                            rows_v.at[sl], [gsem0, gsem1][sl]).wait()
                for g in range(_GB // 16):
                    j0 = b * _GB + g * 16

                    @pl.when(j0 < n_matched)
                    def _(j0=j0, g=g, slot=slot):
                        dvec = dlist[pl.ds(j0, 16)]
                        evec = ewlist[pl.ds(j0, 16)]
                        for l in range(16):

                            @pl.when(j0 + l < n_matched)
                            def _(l=l, dvec=dvec, evec=evec, g=g, slot=slot):
                                dloc = dvec[l]
                                s32 = jnp.broadcast_to(evec[l], (16,))
                                scale = plsc.pack(
                                    s32, s32,
                                    format=plsc.PackFormat.INTERLEAVED)
                                jl = g * 16 + l
                                for q in range(4):
                                    a = acc_v[dloc, pl.ds(q * 16, 16)]
                                    mi = rows_v[slot, jl, pl.ds(q * 16, 16)]
                                    ab = plsc.bitcast(a, jnp.bfloat16)
                                    mb = plsc.bitcast(mi, jnp.bfloat16)
                                    nv = jnp.maximum(ab, mb * scale)
                                    acc_v[dloc, pl.ds(q * 16, 16)] = (
                                        plsc.bitcast(nv, jnp.int32))
                                if with_counts:
                                    plsc.addupdate_scatter(
                                        cnt_v,
                                        [jnp.broadcast_to(dloc, (16,))],
                                        jnp.ones((16,), jnp.float32),
                                        mask=lax.iota(jnp.int32, 16) == 0)
                return 0

            lax.fori_loop(0, nb, batch_body, 0)

        def fire_chunk(c, slot):
            sem = [csem0, csem1][slot]
            pltpu.async_copy(key_hbm.at[pl.ds(c * _CS, _CS)],
                             keyb.at[slot], sem)
            pltpu.async_copy(gidx_hbm.at[pl.ds(c * _CS, _CS)],
                             gixb.at[slot], sem)
            pltpu.async_copy(ew_hbm.at[pl.ds(c * _CS, _CS)],
                             ewb.at[slot], sem)

        fire_chunk(0, 0)

        def chunk_body(c, _):
            cslot = lax.rem(c, 2)

            for sl in range(2):
                @pl.when((c + 1 < n_chunks) & (cslot == sl))
                def _(sl=sl):
                    fire_chunk(c + 1, 1 - sl)

            for sl in range(2):
                @pl.when(cslot == sl)
                def _(sl=sl):
                    sem = [csem0, csem1][sl]
                    pltpu.make_async_copy(
                        key_hbm.at[pl.ds(c * _CS, _CS)], keyb.at[sl],
                        sem).wait()
                    pltpu.make_async_copy(
                        gidx_hbm.at[pl.ds(c * _CS, _CS)], gixb.at[sl],
                        sem).wait()
                    pltpu.make_async_copy(
                        ew_hbm.at[pl.ds(c * _CS, _CS)], ewb.at[sl],
                        sem).wait()

            def vreg_body(i, _):
                offv = offr[...]
                ks = [keyb[cslot, pl.ds((i * 4 + u) * 16, 16)]
                      for u in range(4)]
                locs = [k - base for k in ks]
                inbs = [(lo >= 0) & (lo < _R) for lo in locs]
                one = jnp.ones((16,), jnp.int32)
                zero = jnp.zeros((16,), jnp.int32)
                ibs = [jnp.where(m, one, zero) for m in inbs]
                cums = [plsc.cumsum(b) for b in ibs]
                pcs = [plsc.all_reduce_population_count(m) for m in inbs]
                for u in range(4):
                    pos = offv + cums[u] - 1
                    plsc.store_scatter(
                        glist, [pos],
                        gixb[cslot, pl.ds((i * 4 + u) * 16, 16)],
                        mask=inbs[u])
                    plsc.store_scatter(dlist, [pos], locs[u], mask=inbs[u])
                    plsc.store_scatter(
                        ewlist, [pos],
                        ewb[cslot, pl.ds((i * 4 + u) * 16, 16)],
                        mask=inbs[u])
                    offv = offv + pcs[u]
                offr[...] = offv
                return 0

            lax.fori_loop(0, _CS // 64, vreg_body, 0)
            n_matched = offr[...][0]
            do_drain = n_matched >= (_CAP - _CS)

            @pl.when(do_drain)
            def _():
                drain(n_matched)
                offr[...] = jnp.zeros((16,), jnp.int32)

            return 0

        lax.fori_loop(0, n_chunks, chunk_body, 0)
        drain(offr[...][0])

        pltpu.sync_copy(acc_v, acc_out.at[pl.ds(base, _R)])
        if with_counts:
            pltpu.sync_copy(cnt_v, cnt_out.at[pl.ds(base, _R)])

    return seg_max


def _pack_bf16(x):
    b = x.astype(jnp.bfloat16)
    return jax.lax.bitcast_convert_type(
        b.reshape(x.shape[0], x.shape[1] // 2, 2), jnp.int32)


def _unpack_bf16(p):
    b = jax.lax.bitcast_convert_type(p, jnp.bfloat16)
    return b.reshape(p.shape[0], p.shape[1] * 2)


def _segmax(table_f32, key, gidx, ew, n_out, with_counts=False):
    e = key.shape[0]
    n_chunks = -(-e // _CS)
    e_pad = n_chunks * _CS
    pad = e_pad - e
    key_p = jnp.concatenate([key, jnp.full((pad,), 2**30, jnp.int32)])
    gidx_p = jnp.concatenate([gidx, jnp.zeros((pad,), jnp.int32)])
    ew_p = jnp.concatenate([ew, jnp.zeros((pad,), jnp.float32)])
    table = _pack_bf16(table_f32)
    fn = _make_segmax(table.shape[0], n_chunks, with_counts)
    if with_counts:
        acc, cnt = fn(table, key_p, gidx_p, ew_p)
        return (jax.nn.relu(_unpack_bf16(acc)[:n_out].astype(jnp.float32)),
                cnt[:n_out])
    acc = fn(table, key_p, gidx_p, ew_p)
    return jax.nn.relu(_unpack_bf16(acc)[:n_out].astype(jnp.float32))


def kernel(node_feat, net_feat, pin_feat, edge_feat, gc_W, gc_b, t_pool_W, t_pool_b,
           t_neigh_W, t_self_W, t_self_b, g_pool_W, g_pool_b, g_neigh_W, g_self_W,
           g_self_b, topo_w_W, topo_w_b, geom_w_W, geom_w_b, net_lin_W, net_lin_b,
           pin_src, pin_dst, near_src, near_dst):
    N_CELL, _ = node_feat.shape
    N_NET, _ = net_feat.shape
    E_PIN = pin_src.shape[0]
    E_NEAR = near_src.shape[0]

    ew_pin = jax.nn.sigmoid(pin_feat @ topo_w_W + topo_w_b)
    ew_near = jax.nn.sigmoid(edge_feat @ geom_w_W + geom_w_b)

    h_pool = _relu_mm(net_feat, t_pool_W, t_pool_b, 1000)
    neigh, deg_src = _segmax(h_pool, pin_src, pin_dst, ew_pin[:, 0], N_CELL,
                             with_counts=True)

    h_pool2 = _relu_mm(node_feat, g_pool_W, g_pool_b, 1000)
    neigh2 = _segmax(h_pool2, near_dst, near_src, ew_near[:, 0], N_CELL)

    norm_src = jnp.where(deg_src > 0, deg_src ** -0.5, 0.0)
    feat = node_feat * norm_src[:, None]

    n_net_pad = 10112
    k_chunks = -(-E_PIN // (16 * 128))
    e_pad = 16 * k_chunks * 128
    src_p = jnp.concatenate(
        [pin_src, jnp.zeros((e_pad - E_PIN,), jnp.int32)]).reshape(16, k_chunks, 128)
    dst_p = jnp.concatenate(
        [pin_dst, jnp.full((e_pad - E_PIN,), n_net_pad - 1, jnp.int32)]
    ).reshape(16, k_chunks, 128)
    zc = jnp.zeros((N_CELL, _DAUG - 65), jnp.float32)
    table = jnp.concatenate([
        jnp.concatenate([feat[:, :64], jnp.ones((N_CELL, 1), jnp.float32), zc], 1),
        jnp.concatenate([feat[:, 64:], jnp.zeros((N_CELL, 1), jnp.float32), zc], 1),
    ], axis=0)
    parts = _make_segsum(n_net_pad, k_chunks, N_CELL)(table, src_p, dst_p)
    agg = jnp.concatenate([parts[0, :N_NET, :64], parts[1, :N_NET, :64]], axis=1)
    deg_dst = parts[0, :N_NET, 64]
    norm_dst = jnp.where(deg_dst > 0, deg_dst ** -0.5, 0.0)

    cell_out = _mm3(node_feat, neigh, neigh2,
                    t_self_W + g_self_W, t_neigh_W, g_neigh_W,
                    t_self_b + g_self_b, 1000)

    aggn = agg * norm_dst[:, None]
    net_out = _mm3(aggn, net_feat, jnp.zeros_like(net_feat),
                   gc_W, net_lin_W, jnp.zeros_like(net_lin_W),
                   gc_b + net_lin_b, 1000)
    return (cell_out, net_out)
